# baseline (speedup 1.0000x reference)
import functools

import jax
import jax.numpy as jnp
from jax import lax
from jax.experimental import pallas as pl
from jax.experimental.pallas import tpu as pltpu
from jax.experimental.pallas import tpu_sc as plsc

B = 16384
H = 64
L = 16

_info = plsc.get_sparse_core_info()
NC = _info.num_cores
NS = _info.num_subcores
NW = NC * NS
BPW = B // NW
CH = 128
NCHUNK = BPW // CH

_mesh = plsc.VectorSubcoreMesh(core_axis_name="c", subcore_axis_name="s")


@functools.partial(
    pl.kernel,
    mesh=_mesh,
    compiler_params=pltpu.CompilerParams(use_tc_tiling_on_sc=False),
    out_type=[
        jax.ShapeDtypeStruct((B, H), jnp.float32),
        jax.ShapeDtypeStruct((B, H), jnp.float32),
        jax.ShapeDtypeStruct((B, H), jnp.float32),
    ],
    scratch_types=[
        pltpu.VMEM((CH,), jnp.int32),
        pltpu.VMEM((CH,), jnp.int32),
        pltpu.VMEM((CH, H), jnp.float32),
        pltpu.VMEM((CH, H), jnp.float32),
        pltpu.SemaphoreType.DMA,
    ],
)
def _sc_gather(cids, eids, mf_c, mf_e, mlp_c, mlp_e,
               mf_out, mlpe_out, mlpc_out,
               cid_v, eid_v, buf_a, buf_b, sem):
    wid = lax.axis_index("s") * NC + lax.axis_index("c")
    base = wid * BPW

    def chunk(ch_i, _):
        off = base + ch_i * CH
        pltpu.sync_copy(cids.at[pl.ds(off, CH)], cid_v)
        pltpu.sync_copy(eids.at[pl.ds(off, CH)], eid_v)
        pltpu.async_copy(mlp_e.at[eid_v], buf_a, sem).wait()
        pltpu.sync_copy(buf_a, mlpe_out.at[pl.ds(off, CH)])
        pltpu.async_copy(mlp_c.at[cid_v], buf_a, sem).wait()
        pltpu.sync_copy(buf_a, mlpc_out.at[pl.ds(off, CH)])
        pltpu.async_copy(mf_c.at[cid_v], buf_a, sem).wait()
        pltpu.async_copy(mf_e.at[eid_v], buf_b, sem).wait()

        def mul_row(i, _):
            for j in range(H // L):
                sl = pl.ds(j * L, L)
                buf_a[i, sl] = buf_a[i, sl] * buf_b[i, sl]
            return 0

        lax.fori_loop(0, CH, mul_row, 0)
        pltpu.sync_copy(buf_a, mf_out.at[pl.ds(off, CH)])
        return 0

    lax.fori_loop(0, NCHUNK, chunk, 0)


_TC_BLK = 4096


def _tc_body(mfp, me, mc, w1e, w1c, b1, wmf, wmlp, cb, out):
    h = jnp.dot(me[...], w1e[...], preferred_element_type=jnp.float32)
    h = h + jnp.dot(mc[...], w1c[...], preferred_element_type=jnp.float32)
    h = jnp.maximum(h + b1[...], 0.0)
    z = (jnp.dot(mfp[...], wmf[...], preferred_element_type=jnp.float32)
         + jnp.dot(h, wmlp[...], preferred_element_type=jnp.float32)
         + cb[0, 0])
    out[...] = jax.nn.sigmoid(z)


_TC_BLK = 4096


def _tc_head(mf_prod, mlpe, mlpc, w1e, w1c, b1, wmf, wmlp, cb):
    grid = (B // _TC_BLK,)
    return pl.pallas_call(
        _tc_body,
        grid=grid,
        in_specs=[
            pl.BlockSpec((_TC_BLK, H), lambda i: (i, 0)),
            pl.BlockSpec((_TC_BLK, H), lambda i: (i, 0)),
            pl.BlockSpec((_TC_BLK, H), lambda i: (i, 0)),
            pl.BlockSpec((H, H), lambda i: (0, 0)),
            pl.BlockSpec((H, H), lambda i: (0, 0)),
            pl.BlockSpec((1, H), lambda i: (0, 0)),
            pl.BlockSpec((H, 1), lambda i: (0, 0)),
            pl.BlockSpec((H, 1), lambda i: (0, 0)),
            pl.BlockSpec((1, 1), lambda i: (0, 0)),
        ],
        out_specs=pl.BlockSpec((_TC_BLK, 1), lambda i: (i, 0)),
        out_shape=jax.ShapeDtypeStruct((B, 1), jnp.float32),
    )(mf_prod, mlpe, mlpc, w1e, w1c, b1, wmf, wmlp, cb)


def kernel(compound_ids, enzyme_ids, mf_c, mf_e, mlp_c, mlp_e,
           fc1_w, fc1_b, ce_w, ce_b):
    cids = compound_ids.astype(jnp.int32)
    eids = enzyme_ids.astype(jnp.int32)
    mf_prod, mlpe, mlpc = _sc_gather(cids, eids, mf_c, mf_e, mlp_c, mlp_e)
    w1e = fc1_w[:, :H].T
    w1c = fc1_w[:, H:].T
    b1 = fc1_b.reshape(1, H)
    wmf = ce_w[:, :H].T
    wmlp = ce_w[:, H:].T
    cb = ce_b.reshape(1, 1)
    return _tc_head(mf_prod, mlpe, mlpc, w1e, w1c, b1, wmf, wmlp, cb)


# speedup vs baseline: 1.2309x; 1.2309x over previous
import functools

import jax
import jax.numpy as jnp
from jax import lax
from jax.experimental import pallas as pl
from jax.experimental.pallas import tpu as pltpu
from jax.experimental.pallas import tpu_sc as plsc

B = 16384
H = 64
L = 16

_info = plsc.get_sparse_core_info()
NC = _info.num_cores
NS = _info.num_subcores
NW = NC * NS
BPW = B // NW
CH = 128
NCHUNK = BPW // CH

_mesh = plsc.VectorSubcoreMesh(core_axis_name="c", subcore_axis_name="s")


@functools.partial(
    pl.kernel,
    mesh=_mesh,
    out_type=[
        jax.ShapeDtypeStruct((B, 2 * H), jnp.float32),
        jax.ShapeDtypeStruct((B, 2 * H), jnp.float32),
    ],
    scratch_types=[
        pltpu.VMEM((CH,), jnp.int32),
        pltpu.VMEM((CH,), jnp.int32),
        pltpu.VMEM((CH, 2 * H), jnp.float32),
        pltpu.VMEM((CH, 2 * H), jnp.float32),
        pltpu.SemaphoreType.DMA,
    ],
)
def _sc_gather(cids, eids, cat_c, cat_e,
               outc, oute,
               cid_v, eid_v, bufc, bufe, sem):
    wid = lax.axis_index("s") * NC + lax.axis_index("c")
    base = wid * BPW

    def chunk(ch_i, _):
        off = base + ch_i * CH
        pltpu.sync_copy(cids.at[pl.ds(off, CH)], cid_v)
        pltpu.sync_copy(eids.at[pl.ds(off, CH)], eid_v)
        cpy_c = pltpu.async_copy(cat_c.at[cid_v], bufc, sem)
        cpy_e = pltpu.async_copy(cat_e.at[eid_v], bufe, sem)
        cpy_c.wait()
        cpy_e.wait()

        def mul_row(i, _):
            for j in range(H // L):
                sl = pl.ds(j * L, L)
                bufc[i, sl] = bufc[i, sl] * bufe[i, sl]
            return 0

        lax.fori_loop(0, CH, mul_row, 0)
        pltpu.sync_copy(bufc, outc.at[pl.ds(off, CH)])
        pltpu.sync_copy(bufe, oute.at[pl.ds(off, CH)])
        return 0

    lax.fori_loop(0, NCHUNK, chunk, 0)


_TC_BLK = 4096


def _tc_body(outc, oute, w1e, w1c, b1, wmf, wmlp, cb, out):
    mfp = outc[:, :H]
    mc = outc[:, H:]
    me = oute[:, H:]
    h = jnp.dot(me, w1e[...], preferred_element_type=jnp.float32)
    h = h + jnp.dot(mc, w1c[...], preferred_element_type=jnp.float32)
    h = jnp.maximum(h + b1[...], 0.0)
    z = (jnp.dot(mfp, wmf[...], preferred_element_type=jnp.float32)
         + jnp.dot(h, wmlp[...], preferred_element_type=jnp.float32)
         + cb[0, 0])
    out[...] = jax.nn.sigmoid(z)


def _tc_head(outc, oute, w1e, w1c, b1, wmf, wmlp, cb):
    grid = (B // _TC_BLK,)
    return pl.pallas_call(
        _tc_body,
        grid=grid,
        in_specs=[
            pl.BlockSpec((_TC_BLK, 2 * H), lambda i: (i, 0)),
            pl.BlockSpec((_TC_BLK, 2 * H), lambda i: (i, 0)),
            pl.BlockSpec((H, H), lambda i: (0, 0)),
            pl.BlockSpec((H, H), lambda i: (0, 0)),
            pl.BlockSpec((1, H), lambda i: (0, 0)),
            pl.BlockSpec((H, 1), lambda i: (0, 0)),
            pl.BlockSpec((H, 1), lambda i: (0, 0)),
            pl.BlockSpec((1, 1), lambda i: (0, 0)),
        ],
        out_specs=pl.BlockSpec((_TC_BLK, 1), lambda i: (i, 0)),
        out_shape=jax.ShapeDtypeStruct((B, 1), jnp.float32),
    )(outc, oute, w1e, w1c, b1, wmf, wmlp, cb)


def kernel(compound_ids, enzyme_ids, mf_c, mf_e, mlp_c, mlp_e,
           fc1_w, fc1_b, ce_w, ce_b):
    cids = compound_ids.astype(jnp.int32)
    eids = enzyme_ids.astype(jnp.int32)
    cat_c = jnp.concatenate([mf_c, mlp_c], axis=1)
    cat_e = jnp.concatenate([mf_e, mlp_e], axis=1)
    outc, oute = _sc_gather(cids, eids, cat_c, cat_e)
    w1e = fc1_w[:, :H].T
    w1c = fc1_w[:, H:].T
    b1 = fc1_b.reshape(1, H)
    wmf = ce_w[:, :H].T
    wmlp = ce_w[:, H:].T
    cb = ce_b.reshape(1, 1)
    return _tc_head(outc, oute, w1e, w1c, b1, wmf, wmlp, cb)
